# baseline (speedup 1.0000x reference)
import numpy as np

import jax
import jax.numpy as jnp
from jax import lax
from jax.experimental import pallas as pl
from jax.experimental.pallas import tpu as pltpu
from jax.experimental.pallas import tpu_sc as plsc

_ALPHA = 0.2
_LAM = float(np.random.RandomState(0).beta(_ALPHA, _ALPHA))

_NSUB = 16
_GPR = 128 // _NSUB
_ROWS_PER_W = 2

with jax.default_device(jax.devices("cpu")[0]):
    _PERM_NP = np.asarray(
        jax.random.permutation(jax.random.key(42), 64)).astype(np.int32)


def _sc_mix_body(x_hbm, out_hbm, a0, a1, b0, b1, o0, o1, sa, sb, so):
    nc = 2
    wid = lax.axis_index("s") * nc + lax.axis_index("c")

    abufs = (a0, a1)
    bbufs = (b0, b1)
    obufs = (o0, o1)

    rows = []
    for k in range(_ROWS_PER_W):
        r = wid * _ROWS_PER_W + k
        q = jnp.int32(_PERM_NP[k])
        for w in range(32):
            q = jnp.where(wid == w, jnp.int32(_PERM_NP[w * _ROWS_PER_W + k]), q)
        rows.append((r, q))

    steps = [(k, g) for k in range(_ROWS_PER_W) for g in range(_GPR)]
    n = len(steps)

    def issue_in(gg):
        k, g = steps[gg]
        r, q = rows[k]
        ha = pltpu.async_copy(
            x_hbm.at[r, pl.ds(g * _NSUB, _NSUB)], abufs[gg % 2], sa.at[gg % 2])
        hb = pltpu.async_copy(
            x_hbm.at[q, pl.ds(g * _NSUB, _NSUB)], bbufs[gg % 2], sb.at[gg % 2])
        return (ha, hb)

    def issue_out(gg):
        k, g = steps[gg]
        r, _ = rows[k]
        return pltpu.async_copy(
            obufs[gg % 2], out_hbm.at[r, pl.ds(g * _NSUB, _NSUB)], so.at[gg % 2])

    in_h = [None] * n
    out_h = [None] * n
    in_h[0] = issue_in(0)

    for gg in range(n):
        if gg + 1 < n:
            in_h[gg + 1] = issue_in(gg + 1)
        in_h[gg][0].wait()
        in_h[gg][1].wait()
        if gg >= 2:
            out_h[gg - 2].wait()
        a, b, o = abufs[gg % 2], bbufs[gg % 2], obufs[gg % 2]

        def blend(i, _):
            s = i // 64
            col = (i % 64) * 16
            sl = pl.ds(col, 16)
            o[s, sl] = _LAM * a[s, sl] + (1.0 - _LAM) * b[s, sl]
            return 0

        lax.fori_loop(0, _NSUB * 64, blend, 0)
        out_h[gg] = issue_out(gg)

    out_h[n - 2].wait()
    out_h[n - 1].wait()


def _lab_kernel(perm_ref, l_ref, ol_ref):
    i = pl.program_id(0)
    j = perm_ref[i]
    ol_ref[0, 0] = _LAM * l_ref[i, 0] + (1.0 - _LAM) * l_ref[j, 0]


def kernel(batch_spectrograms, batch_labels):
    B, C, H, W = batch_spectrograms.shape
    L = batch_labels.shape[1]
    perm = jax.random.permutation(jax.random.key(42), B).astype(jnp.int32)

    x3 = batch_spectrograms.reshape(B, H, W)

    mesh = plsc.VectorSubcoreMesh(core_axis_name="c", subcore_axis_name="s")
    sc_call = pl.kernel(
        _sc_mix_body,
        mesh=mesh,
        out_type=jax.ShapeDtypeStruct((B, H, W), jnp.float32),
        scratch_types=[
            pltpu.VMEM((_NSUB, W), jnp.float32),
            pltpu.VMEM((_NSUB, W), jnp.float32),
            pltpu.VMEM((_NSUB, W), jnp.float32),
            pltpu.VMEM((_NSUB, W), jnp.float32),
            pltpu.VMEM((_NSUB, W), jnp.float32),
            pltpu.VMEM((_NSUB, W), jnp.float32),
            pltpu.SemaphoreType.DMA((2,)),
            pltpu.SemaphoreType.DMA((2,)),
            pltpu.SemaphoreType.DMA((2,)),
        ],
    )
    ox = sc_call(x3).reshape(B, C, H, W)

    labels3 = batch_labels[:, None, :]
    grid_spec = pltpu.PrefetchScalarGridSpec(
        num_scalar_prefetch=1,
        grid=(B,),
        in_specs=[pl.BlockSpec((B, 1, L), lambda g, p: (0, 0, 0))],
        out_specs=[pl.BlockSpec((1, 1, L), lambda g, p: (g, 0, 0))],
    )
    ol = pl.pallas_call(
        _lab_kernel,
        grid_spec=grid_spec,
        out_shape=[jax.ShapeDtypeStruct(labels3.shape, jnp.float32)],
    )(perm, labels3)[0]
    return ox, ol[:, 0, :]


# speedup vs baseline: 1.5257x; 1.5257x over previous
import numpy as np

import jax
import jax.numpy as jnp
from jax import lax
from jax.experimental import pallas as pl
from jax.experimental.pallas import tpu as pltpu
from jax.experimental.pallas import tpu_sc as plsc

_ALPHA = 0.2
_LAM = float(np.random.RandomState(0).beta(_ALPHA, _ALPHA))

_NSUB = 16
_GPR = 128 // _NSUB
_ROWS_PER_W = 2

with jax.default_device(jax.devices("cpu")[0]):
    _PERM_NP = np.asarray(
        jax.random.permutation(jax.random.key(42), 64)).astype(np.int32)


def _sc_mix_body(x_hbm, out_hbm, a0, a1, b0, b1, o0, o1, sa, sb, so):
    nc = 2
    wid = lax.axis_index("s") * nc + lax.axis_index("c")

    abufs = (a0, a1)
    bbufs = (b0, b1)
    obufs = (o0, o1)

    rows = []
    for k in range(_ROWS_PER_W):
        r = wid * _ROWS_PER_W + k
        q = jnp.int32(_PERM_NP[k])
        for w in range(32):
            q = jnp.where(wid == w, jnp.int32(_PERM_NP[w * _ROWS_PER_W + k]), q)
        rows.append((r, q))

    steps = [(k, g) for k in range(_ROWS_PER_W) for g in range(_GPR)]
    n = len(steps)

    def issue_in(gg):
        k, g = steps[gg]
        r, q = rows[k]
        ha = pltpu.async_copy(
            x_hbm.at[r, pl.ds(g * _NSUB, _NSUB)], abufs[gg % 2], sa.at[gg % 2])
        hb = pltpu.async_copy(
            x_hbm.at[q, pl.ds(g * _NSUB, _NSUB)], bbufs[gg % 2], sb.at[gg % 2])
        return (ha, hb)

    def issue_out(gg):
        k, g = steps[gg]
        r, _ = rows[k]
        return pltpu.async_copy(
            obufs[gg % 2], out_hbm.at[r, pl.ds(g * _NSUB, _NSUB)], so.at[gg % 2])

    in_h = [None] * n
    out_h = [None] * n
    in_h[0] = issue_in(0)

    for gg in range(n):
        if gg + 1 < n:
            in_h[gg + 1] = issue_in(gg + 1)
        in_h[gg][0].wait()
        in_h[gg][1].wait()
        if gg >= 2:
            out_h[gg - 2].wait()
        a, b, o = abufs[gg % 2], bbufs[gg % 2], obufs[gg % 2]

        @plsc.parallel_loop(0, _NSUB * 64, unroll=8)
        def _blend(i):
            s = i // 64
            col = (i % 64) * 16
            sl = pl.ds(col, 16)
            o[s, sl] = _LAM * a[s, sl] + (1.0 - _LAM) * b[s, sl]
        out_h[gg] = issue_out(gg)

    out_h[n - 2].wait()
    out_h[n - 1].wait()


def _lab_kernel(perm_ref, l_ref, ol_ref):
    i = pl.program_id(0)
    j = perm_ref[i]
    ol_ref[0, 0] = _LAM * l_ref[i, 0] + (1.0 - _LAM) * l_ref[j, 0]


def kernel(batch_spectrograms, batch_labels):
    B, C, H, W = batch_spectrograms.shape
    L = batch_labels.shape[1]
    perm = jax.random.permutation(jax.random.key(42), B).astype(jnp.int32)

    x3 = batch_spectrograms.reshape(B, H, W)

    mesh = plsc.VectorSubcoreMesh(core_axis_name="c", subcore_axis_name="s")
    sc_call = pl.kernel(
        _sc_mix_body,
        mesh=mesh,
        out_type=jax.ShapeDtypeStruct((B, H, W), jnp.float32),
        scratch_types=[
            pltpu.VMEM((_NSUB, W), jnp.float32),
            pltpu.VMEM((_NSUB, W), jnp.float32),
            pltpu.VMEM((_NSUB, W), jnp.float32),
            pltpu.VMEM((_NSUB, W), jnp.float32),
            pltpu.VMEM((_NSUB, W), jnp.float32),
            pltpu.VMEM((_NSUB, W), jnp.float32),
            pltpu.SemaphoreType.DMA((2,)),
            pltpu.SemaphoreType.DMA((2,)),
            pltpu.SemaphoreType.DMA((2,)),
        ],
    )
    ox = sc_call(x3).reshape(B, C, H, W)

    labels3 = batch_labels[:, None, :]
    grid_spec = pltpu.PrefetchScalarGridSpec(
        num_scalar_prefetch=1,
        grid=(B,),
        in_specs=[pl.BlockSpec((B, 1, L), lambda g, p: (0, 0, 0))],
        out_specs=[pl.BlockSpec((1, 1, L), lambda g, p: (g, 0, 0))],
    )
    ol = pl.pallas_call(
        _lab_kernel,
        grid_spec=grid_spec,
        out_shape=[jax.ShapeDtypeStruct(labels3.shape, jnp.float32)],
    )(perm, labels3)[0]
    return ox, ol[:, 0, :]
